# baseline (speedup 1.0000x reference)
import functools

import jax
import jax.numpy as jnp
from jax import lax
from jax.experimental import pallas as pl
from jax.experimental.pallas import tpu as pltpu
from jax.experimental.pallas import tpu_sc as plsc

B = 128
V = 100000
CH = 128
CF = 781
C = 782
CT = 784
K = 50
ROWS = 8
NEG = float("-inf")
BIGI = 2**30


def _k1a_body(x_ref, m_ref, t_ref):
    x = x_ref[...]
    body = jnp.max(x[:, :CF * CH].reshape(ROWS, CF, CH), axis=-1)
    tail = jnp.max(x[:, CF * CH:], axis=-1)
    m_ref[:, :CF] = body
    m_ref[:, CF:] = tail[:, None]
    pad = jnp.full((ROWS, CT * CH - V), NEG, dtype=jnp.float32)
    xp = jnp.concatenate([x, pad], axis=1)
    t_ref[...] = xp.reshape(ROWS * CT, CH)


def _k1a(logits):
    return pl.pallas_call(
        _k1a_body,
        grid=(B // ROWS,),
        in_specs=[pl.BlockSpec((ROWS, V), lambda i: (i, 0))],
        out_specs=[pl.BlockSpec((ROWS, C), lambda i: (i, 0)),
                   pl.BlockSpec((ROWS * CT, CH), lambda i: (i, 0))],
        out_shape=[jax.ShapeDtypeStruct((B, C), jnp.float32),
                   jax.ShapeDtypeStruct((B * CT, CH), jnp.float32)],
    )(logits)


def _k1b_body(m_ref, cids_ref, flat_ref):
    m = m_ref[...]
    iota_c = lax.broadcasted_iota(jnp.int32, (B, C), 1)
    sel = []
    for _ in range(K):
        best = jnp.max(m, axis=1)
        eq = m == best[:, None]
        bidx = jnp.min(jnp.where(eq, iota_c, BIGI), axis=1)
        sel.append(bidx)
        m = jnp.where(iota_c == bidx[:, None], NEG, m)
    s = jnp.concatenate([b[:, None] for b in sel], axis=1)
    ranks = jnp.sum((s[:, None, :] < s[:, :, None]).astype(jnp.int32), axis=2)
    iota_p = lax.broadcasted_iota(jnp.int32, (B, K, K), 2)
    sorted_s = jnp.sum(jnp.where(ranks[:, :, None] == iota_p,
                                 s[:, :, None], 0), axis=1)
    cids_ref[...] = sorted_s
    rows = lax.broadcasted_iota(jnp.int32, (B, 1), 0)
    flat_ref[...] = sorted_s + rows * CT


def _k1b(m):
    return pl.pallas_call(
        _k1b_body,
        out_shape=[jax.ShapeDtypeStruct((B, K), jnp.int32),
                   jax.ShapeDtypeStruct((B, K), jnp.int32)],
    )(m)


def _sc_gather(table, idx2d, width, tc_tiling=True):
    info = plsc.get_sparse_core_info()
    nw = info.num_cores * info.num_subcores
    npart, part = idx2d.shape
    npart //= nw
    total = nw * npart * part
    per_w = total // nw
    mesh = plsc.VectorSubcoreMesh(core_axis_name="c", subcore_axis_name="s")

    @functools.partial(
        pl.kernel, mesh=mesh,
        out_type=jax.ShapeDtypeStruct((total, width), jnp.float32),
        scratch_types=[
            pltpu.VMEM((npart, part), jnp.int32),
            pltpu.VMEM((per_w, width), jnp.float32),
            pltpu.SemaphoreType.DMA,
        ],
        compiler_params=pltpu.CompilerParams(use_tc_tiling_on_sc=tc_tiling),
    )
    def k(table_hbm, idx_hbm, out_hbm, idx_v, rows_v, sem):
        wid = lax.axis_index("s") * info.num_cores + lax.axis_index("c")
        pltpu.sync_copy(idx_hbm.at[pl.ds(wid * npart, npart)], idx_v)
        for j in range(npart):
            pltpu.async_copy(table_hbm.at[idx_v.at[j]],
                             rows_v.at[pl.ds(j * part, part)], sem).wait()
        pltpu.sync_copy(rows_v, out_hbm.at[pl.ds(wid * per_w, per_w)])

    return k(table, idx2d)


def _rotl(x, d):
    return (x << jnp.uint32(d)) | (x >> jnp.uint32(32 - d))


def _threefry_bits(c1):
    ks0 = jnp.uint32(0)
    ks1 = jnp.uint32(12345)
    ks2 = ks0 ^ ks1 ^ jnp.uint32(0x1BD11BDA)
    ks = [ks0, ks1, ks2]
    x0 = jnp.zeros_like(c1) + ks0
    x1 = c1 + ks1
    rots = [[13, 15, 26, 6], [17, 29, 16, 24]]
    for i in range(5):
        for r in rots[i % 2]:
            x0 = x0 + x1
            x1 = _rotl(x1, r)
            x1 = x0 ^ x1
        x0 = x0 + ks[(i + 1) % 3]
        x1 = x1 + ks[(i + 2) % 3] + jnp.uint32(i + 1)
    return x0 ^ x1


CL = 16
GPC = CH // CL
G = K * GPC
K2 = 52
LOW = -1e30


def _d1a_body(cand_ref, cm_ref):
    v = jnp.maximum(cand_ref[...], LOW)
    t = v
    for sh in (8, 4, 2, 1):
        t = jnp.maximum(t, pltpu.roll(t, CH - sh, 1))
    l_i = lax.broadcasted_iota(jnp.int32, (CH, GPC), 0)
    g_i = lax.broadcasted_iota(jnp.int32, (CH, GPC), 1)
    sel = (l_i == g_i * CL).astype(jnp.float32)
    cm_ref[...] = lax.dot_general(t, sel, (((1,), (0,)), ((), ())),
                                  preferred_element_type=jnp.float32)


def _d1a(cand2d):
    return pl.pallas_call(
        _d1a_body,
        out_shape=jax.ShapeDtypeStruct((B * K, GPC), jnp.float32),
    )(cand2d)


def _d1_body(cm_ref, mids_ref, flat2_ref):
    cm = cm_ref[...]
    iota_g = lax.broadcasted_iota(jnp.int32, (B, G), 1)
    selids = []
    for _ in range(K2):
        best = jnp.max(cm, axis=1)
        eq = cm == best[:, None]
        bidx = jnp.min(jnp.where(eq, iota_g, BIGI), axis=1)
        selids.append(bidx)
        cm = jnp.where(iota_g == bidx[:, None], NEG, cm)
    s = jnp.concatenate([b[:, None] for b in selids], axis=1)
    ranks = jnp.sum((s[:, None, :] < s[:, :, None]).astype(jnp.int32), axis=2)
    iota_p = lax.broadcasted_iota(jnp.int32, (B, K2, K2), 2)
    sorted_s = jnp.sum(jnp.where(ranks[:, :, None] == iota_p,
                                 s[:, :, None], 0), axis=1)
    mids_ref[...] = sorted_s
    rows = lax.broadcasted_iota(jnp.int32, (B, 1), 0)
    flat2_ref[...] = sorted_s + rows * G


def _d1(cm400):
    return pl.pallas_call(
        _d1_body,
        out_shape=[jax.ShapeDtypeStruct((B, K2), jnp.int32),
                   jax.ShapeDtypeStruct((B, K2), jnp.int32)],
    )(cm400)


def _d2_body(c2_ref, mids_ref, cids_ref, temp_ref, tok_ref):
    v = c2_ref[...]
    mids = mids_ref[...]
    cids = cids_ref[...]
    temp = temp_ref[...]
    ks = mids // GPC
    sub = mids - ks * GPC
    iota_k = lax.broadcasted_iota(jnp.int32, (B, K2, K), 2)
    cbase = jnp.sum(jnp.where(ks[:, :, None] == iota_k,
                              cids[:, None, :], 0), axis=2)
    cols = ((cbase * CH + sub * CL)[:, :, None]
            + lax.broadcasted_iota(jnp.int32, (B, K2, CL), 2))
    selv, selc = [], []
    for _ in range(K):
        m1 = jnp.max(v, axis=2)
        best = jnp.max(m1, axis=1)
        eq = v == best[:, None, None]
        c1 = jnp.min(jnp.where(eq, cols, BIGI), axis=2)
        bcol = jnp.min(c1, axis=1)
        selv.append(best)
        selc.append(bcol)
        v = jnp.where(cols == bcol[:, None, None], NEG, v)
    sv = jnp.concatenate([b[:, None] for b in selv], axis=1)
    sc = jnp.concatenate([b[:, None] for b in selc], axis=1)
    rows = lax.broadcasted_iota(jnp.int32, (B, 1), 0)
    flat = (rows * V + sc).astype(jnp.uint32)
    bits = _threefry_bits(flat)
    u = lax.bitcast_convert_type((bits >> jnp.uint32(9)) | jnp.uint32(0x3F800000),
                                 jnp.float32) - jnp.float32(1.0)
    noise = jnp.maximum(-jnp.log1p(-u), jnp.float32(1e-10))
    score = sv / temp - jnp.log(noise)
    bs = jnp.max(score, axis=1)
    tok = jnp.min(jnp.where(score == bs[:, None], sc, BIGI), axis=1)
    tok_ref[...] = tok[:, None]


def _d2(c2, mids, cids, temps2):
    return pl.pallas_call(
        _d2_body,
        out_shape=jax.ShapeDtypeStruct((B, 1), jnp.int32),
    )(c2, mids, cids, temps2)


def kernel(logits, temperatures, top_k, top_p):
    del top_k, top_p
    logits = logits.astype(jnp.float32)
    m, table = _k1a(logits)
    cids, flat = _k1b(m)
    cand = _sc_gather(table, flat.reshape(64, 100), CH)
    cm = _d1a(cand)
    mids, flat2 = _d1(cm.reshape(B, G))
    cand2 = _sc_gather(cand.reshape(B * G, CL),
                       flat2.reshape(64, 2 * K2), CL,
                       tc_tiling=False)
    tok = _d2(cand2.reshape(B, K2, CL), mids, cids,
              temperatures.reshape(B, 1))
    return tok.reshape(B).astype(jnp.int32)
